# baseline (speedup 1.0000x reference)
import jax
import jax.numpy as jnp
from jax.experimental import pallas as pl
from jax.experimental.pallas import tpu as pltpu

_BM = 200


def _fused_kernel(x_ref, w_ref, a_ref, b_ref, out_ref, s_ref):
    @pl.when(pl.program_id(0) == 0)
    def _():
        s_ref[...] = jnp.dot(
            x_ref[...], w_ref[...], preferred_element_type=jnp.float32)

    s = s_ref[...]
    out_ref[pl.ds(0, _BM), :] = jnp.dot(
        a_ref[...], s, preferred_element_type=jnp.float32)
    out_ref[pl.ds(_BM, _BM), :] = jnp.dot(
        b_ref[...], s, preferred_element_type=jnp.float32)


def kernel(input, adj, weight):
    n, f_in = input.shape
    f_out = weight.shape[1]
    return pl.pallas_call(
        _fused_kernel,
        grid=(n // (2 * _BM),),
        in_specs=[
            pl.BlockSpec((n, f_in), lambda i: (0, 0)),
            pl.BlockSpec((f_in, f_out), lambda i: (0, 0)),
            pl.BlockSpec((_BM, n), lambda i: (2 * i, 0)),
            pl.BlockSpec((_BM, n), lambda i: (2 * i + 1, 0)),
        ],
        out_specs=pl.BlockSpec((2 * _BM, f_out), lambda i: (i, 0)),
        out_shape=jax.ShapeDtypeStruct((n, f_out), jnp.float32),
        scratch_shapes=[pltpu.VMEM((n, f_out), jnp.float32)],
    )(input, weight, adj, adj)


# speedup vs baseline: 1.0087x; 1.0087x over previous
import jax
import jax.numpy as jnp
from jax.experimental import pallas as pl
from jax.experimental.pallas import tpu as pltpu

_BM = 400


def _fused_kernel(x_ref, w_ref, adj_ref, out_ref, s_ref):
    @pl.when(pl.program_id(0) == 0)
    def _():
        s_ref[...] = jnp.dot(
            x_ref[...],
            w_ref[...],
            preferred_element_type=jnp.float32,
        )

    out_ref[...] = jnp.dot(
        adj_ref[...],
        s_ref[...],
        preferred_element_type=jnp.float32,
    )


def kernel(input, adj, weight):
    n, f_in = input.shape
    f_out = weight.shape[1]
    return pl.pallas_call(
        _fused_kernel,
        grid=(pl.cdiv(n, _BM),),
        in_specs=[
            pl.BlockSpec((n, f_in), lambda i: (0, 0)),
            pl.BlockSpec((f_in, f_out), lambda i: (0, 0)),
            pl.BlockSpec((_BM, n), lambda i: (i, 0)),
        ],
        out_specs=pl.BlockSpec((_BM, f_out), lambda i: (i, 0)),
        out_shape=jax.ShapeDtypeStruct((n, f_out), jnp.float32),
        scratch_shapes=[pltpu.VMEM((n, f_out), jnp.float32)],
    )(input, weight, adj)
